# baseline (speedup 1.0000x reference)
import numpy as np

import jax
import jax.numpy as jnp
from jax.experimental import pallas as pl
from jax.experimental.pallas import tpu as pltpu

_OFFSETS = tuple(
    [dx for dx in range(1, 5)]
    + [dy * 32 + dx for dy in range(1, 5) for dx in range(-4, 5)
       if dx * dx + dy * dy < 25]
)

_NPIX = 1024
_NSPAN = 892
_NK = len(_OFFSETS)


def _resize_matrix() -> np.ndarray:
    w = np.zeros((32, 64), np.float64)
    for i in range(32):
        for j in range(2 * i - 1, 2 * i + 3):
            if 0 <= j < 64:
                w[i, j] = 1.0 - abs(j - (2 * i + 0.5)) / 2.0
        w[i] /= w[i].sum()
    return w


_R = _resize_matrix()


def _elu(v):
    return jnp.where(v > 0, v, jnp.exp(v) - 1.0)


def _body(x0_ref, x1_ref, x2_ref, w2_ref, rt_ref, w3_ref, w4_ref,
          w9a_ref, w9b_ref, w9c_ref, out_ref):
    f32 = jnp.float32
    bf16 = jnp.bfloat16

    f2 = jnp.dot(w2_ref[...], x0_ref[0].astype(bf16),
                 preferred_element_type=f32)
    f2 = _elu(f2)
    pieces = []
    for i in range(32):
        comb = None
        for h in range(max(0, 2 * i - 1), min(64, 2 * i + 3)):
            term = _R[i, h] * f2[:, h * 64:(h + 1) * 64]
            comb = term if comb is None else comb + term
        pieces.append(jnp.dot(comb.astype(bf16), rt_ref[...],
                              preferred_element_type=f32))
    f2r = jnp.concatenate(pieces, axis=1).astype(bf16)

    f3 = _elu(jnp.dot(w3_ref[...], x1_ref[0].astype(bf16),
                      preferred_element_type=f32)).astype(bf16)
    f4 = _elu(jnp.dot(w4_ref[...], x2_ref[0].astype(bf16),
                      preferred_element_type=f32)).astype(bf16)
    acc = jnp.dot(w9a_ref[...], f2r, preferred_element_type=f32)
    acc += jnp.dot(w9b_ref[...], f3, preferred_element_type=f32)
    acc += jnp.dot(w9c_ref[...], f4, preferred_element_type=f32)
    x = _elu(acc).astype(bf16)

    anchor = x[:, 0:_NSPAN]
    scale = jnp.full((1, x.shape[0]), 1.0 / x.shape[0], dtype=bf16)
    for k, dk in enumerate(_OFFSETS):
        d = jnp.abs(x[:, dk:dk + _NSPAN] - anchor)
        e = jnp.dot(scale, d, preferred_element_type=f32)
        out_ref[0, k, 0:_NSPAN] = jnp.exp(-e)[0]


def kernel(d2_0, d2_1, d2_2, w2, w3, w4, w9):
    B = d2_0.shape[0]
    f32 = jnp.float32
    bf16 = jnp.bfloat16
    X0 = d2_0.reshape(B, 512, 4096)
    X1 = d2_1.reshape(B, 1024, _NPIX)
    X2 = d2_2.reshape(B, 2048, _NPIX)
    RT = jnp.asarray(_R.T, dtype=bf16)

    aff_full = pl.pallas_call(
        _body,
        grid=(B,),
        in_specs=[
            pl.BlockSpec((1, 512, 4096), lambda b: (b, 0, 0)),
            pl.BlockSpec((1, 1024, _NPIX), lambda b: (b, 0, 0)),
            pl.BlockSpec((1, 2048, _NPIX), lambda b: (b, 0, 0)),
            pl.BlockSpec((64, 512), lambda b: (0, 0)),
            pl.BlockSpec((64, 32), lambda b: (0, 0)),
            pl.BlockSpec((128, 1024), lambda b: (0, 0)),
            pl.BlockSpec((320, 2048), lambda b: (0, 0)),
            pl.BlockSpec((512, 64), lambda b: (0, 0)),
            pl.BlockSpec((512, 128), lambda b: (0, 0)),
            pl.BlockSpec((512, 320), lambda b: (0, 0)),
        ],
        out_specs=pl.BlockSpec((1, _NK, 896), lambda b: (b, 0, 0)),
        out_shape=jax.ShapeDtypeStruct((B, _NK, 896), f32),
        compiler_params=pltpu.CompilerParams(
            dimension_semantics=("parallel",)),
    )(X0, X1, X2, w2.astype(bf16), RT, w3.astype(bf16), w4.astype(bf16),
      w9[:, 0:64].astype(bf16), w9[:, 64:192].astype(bf16),
      w9[:, 192:512].astype(bf16))

    aff = aff_full.reshape(B, _NK, 28, 32)[:, :, :, 4:28]
    return aff.reshape(B, _NK, 672)


# speedup vs baseline: 1.0688x; 1.0688x over previous
import numpy as np

import jax
import jax.numpy as jnp
from jax import lax
from jax.experimental import pallas as pl
from jax.experimental.pallas import tpu as pltpu

_OFFSETS = tuple(
    [dx for dx in range(1, 5)]
    + [dy * 32 + dx for dy in range(1, 5) for dx in range(-4, 5)
       if dx * dx + dy * dy < 25]
)

_NPIX = 1024
_NSPAN = 892
_NK = len(_OFFSETS)


def _resize_matrix() -> np.ndarray:
    w = np.zeros((32, 64), np.float64)
    for i in range(32):
        for j in range(2 * i - 1, 2 * i + 3):
            if 0 <= j < 64:
                w[i, j] = 1.0 - abs(j - (2 * i + 0.5)) / 2.0
        w[i] /= w[i].sum()
    return w


_R = _resize_matrix()

_DN_T = (((0,), (1,)), ((), ()))


def _elu(v):
    return jnp.where(v > 0, v, jnp.exp(v) - 1.0)


def _body(x0_ref, x1_ref, x2_ref, w2_ref, r_ref, w3_ref, w4_ref,
          w9at_ref, w9bt_ref, w9ct_ref, out_ref):
    f32 = jnp.float32
    bf16 = jnp.bfloat16

    f2t = lax.dot_general(x0_ref[0].astype(bf16), w2_ref[...], _DN_T,
                          preferred_element_type=f32)
    f2t = _elu(f2t)
    pieces = []
    for i in range(32):
        comb = None
        for h in range(max(0, 2 * i - 1), min(64, 2 * i + 3)):
            term = _R[i, h] * f2t[h * 64:(h + 1) * 64, :]
            comb = term if comb is None else comb + term
        pieces.append(jnp.dot(r_ref[...], comb.astype(bf16),
                              preferred_element_type=f32))
    f2rt = jnp.concatenate(pieces, axis=0).astype(bf16)

    f3t = _elu(lax.dot_general(x1_ref[0].astype(bf16), w3_ref[...], _DN_T,
                               preferred_element_type=f32)).astype(bf16)
    f4t = _elu(lax.dot_general(x2_ref[0].astype(bf16), w4_ref[...], _DN_T,
                               preferred_element_type=f32)).astype(bf16)
    acc = jnp.dot(f2rt, w9at_ref[...], preferred_element_type=f32)
    acc += jnp.dot(f3t, w9bt_ref[...], preferred_element_type=f32)
    acc += jnp.dot(f4t, w9ct_ref[...], preferred_element_type=f32)
    xt = _elu(acc).astype(bf16)

    anchor = xt[0:_NSPAN, :]
    onescol = jnp.full((xt.shape[1], 8), 1.0 / xt.shape[1], dtype=bf16)
    for k, dk in enumerate(_OFFSETS):
        d = jnp.abs(xt[dk:dk + _NSPAN, :] - anchor)
        e = jnp.dot(d, onescol, preferred_element_type=f32)
        out_ref[0, 0:_NSPAN, k:k + 1] = jnp.exp(-e[:, 0:1])


def kernel(d2_0, d2_1, d2_2, w2, w3, w4, w9):
    B = d2_0.shape[0]
    f32 = jnp.float32
    bf16 = jnp.bfloat16
    X0 = d2_0.reshape(B, 512, 4096)
    X1 = d2_1.reshape(B, 1024, _NPIX)
    X2 = d2_2.reshape(B, 2048, _NPIX)
    R = jnp.asarray(_R, dtype=bf16)

    aff_t = pl.pallas_call(
        _body,
        grid=(B,),
        in_specs=[
            pl.BlockSpec((1, 512, 4096), lambda b: (b, 0, 0)),
            pl.BlockSpec((1, 1024, _NPIX), lambda b: (b, 0, 0)),
            pl.BlockSpec((1, 2048, _NPIX), lambda b: (b, 0, 0)),
            pl.BlockSpec((64, 512), lambda b: (0, 0)),
            pl.BlockSpec((32, 64), lambda b: (0, 0)),
            pl.BlockSpec((128, 1024), lambda b: (0, 0)),
            pl.BlockSpec((320, 2048), lambda b: (0, 0)),
            pl.BlockSpec((64, 512), lambda b: (0, 0)),
            pl.BlockSpec((128, 512), lambda b: (0, 0)),
            pl.BlockSpec((320, 512), lambda b: (0, 0)),
        ],
        out_specs=pl.BlockSpec((1, 896, _NK), lambda b: (b, 0, 0)),
        out_shape=jax.ShapeDtypeStruct((B, 896, _NK), f32),
    )(X0, X1, X2, w2.astype(bf16), R, w3.astype(bf16), w4.astype(bf16),
      w9[:, 0:64].T.astype(bf16), w9[:, 64:192].T.astype(bf16),
      w9[:, 192:512].T.astype(bf16))

    aff = aff_t.reshape(B, 28, 32, _NK)[:, :, 4:28, :]
    return aff.transpose(0, 3, 1, 2).reshape(B, _NK, 672)
